# Initial kernel scaffold; baseline (speedup 1.0000x reference)
import jax
import jax.numpy as jnp
from jax.experimental import pallas as pl


def kernel(sequence, token_table):
    raise NotImplementedError("write your pallas kernel here")


# speedup vs baseline: 1.0973x; 1.0973x over previous
import functools

import numpy as np
import jax
import jax.numpy as jnp
from jax import lax
from jax.experimental import pallas as pl
from jax.experimental.pallas import tpu as pltpu
from jax.experimental.pallas import tpu_sc as plsc

SEQ = 200
D = 32
BATCH = 4096
NC = 2
NS = 16
NW = NC * NS
ROWS = BATCH * SEQ
ROWS_PER_W = ROWS // NW
CHUNK = SEQ
NCH = ROWS_PER_W // CHUNK
G = 5
GROUP = CHUNK // G
NBLK = -(-CHUNK // 16)


def _positional_embedding():
    pos = np.arange(SEQ, dtype=np.float32)[:, None]
    exp_sin = np.arange(0, D, 2, dtype=np.float32) / D * 2.0
    exp_cos = np.arange(1, D + 1, 2, dtype=np.float32) / D * 2.0
    sins = np.sin(pos / np.power(10000.0, exp_sin))
    coss = np.cos(pos / np.power(10000.0, exp_cos))
    return np.stack([sins, coss], axis=2).reshape(SEQ, D)


_PE = _positional_embedding()


@functools.partial(
    pl.kernel,
    mesh=plsc.VectorSubcoreMesh(core_axis_name="c", subcore_axis_name="s"),
    out_type=jax.ShapeDtypeStruct((ROWS, D), jnp.float32),
    scratch_types=[
        pltpu.VMEM((CHUNK,), jnp.int32),
        pltpu.VMEM((CHUNK,), jnp.int32),
        pltpu.VMEM((CHUNK, 4 * D), jnp.float32),
        pltpu.VMEM((CHUNK, D), jnp.float32),
        pltpu.VMEM((SEQ, D), jnp.float32),
        pltpu.SemaphoreType.DMA,
    ],
)
def _embed(tok_hbm, table2_hbm, pe_hbm, out_hbm, tok_v, gidx_v, groups_v,
           out_v, pe_v, sem):
    wid = lax.axis_index("s") * NC + lax.axis_index("c")
    pltpu.sync_copy(pe_hbm, pe_v)

    def chunk_body(c, carry):
        base = (wid * NCH + c) * CHUNK
        pltpu.sync_copy(tok_hbm.at[pl.ds(base, CHUNK)], tok_v)

        def idx_body(k, carry2):
            o = lax.min(k * 16, CHUNK - 16)
            gidx_v[pl.ds(o, 16)] = lax.shift_right_logical(
                tok_v[pl.ds(o, 16)], 2)
            return carry2

        lax.fori_loop(0, NBLK, idx_body, 0)

        handles = [
            pltpu.async_copy(
                table2_hbm.at[gidx_v.at[pl.ds(g * GROUP, GROUP)]],
                groups_v.at[pl.ds(g * GROUP, GROUP)],
                sem,
            )
            for g in range(G)
        ]
        for h in handles:
            h.wait()

        def blk_body(b, carry2):
            r0 = lax.min(b * 16, CHUNK - 16)
            tokv = tok_v[pl.ds(r0, 16)]
            for j in range(16):
                r = r0 + j
                off = (tokv[j] & 3) * D
                for h2 in range(D // 16):
                    out_v[r, pl.ds(h2 * 16, 16)] = (
                        groups_v[r, pl.ds(off + h2 * 16, 16)]
                        + pe_v[r, pl.ds(h2 * 16, 16)]
                    )
            return carry2

        lax.fori_loop(0, NBLK, blk_body, 0)
        pltpu.sync_copy(out_v, out_hbm.at[pl.ds(base, CHUNK)])
        return carry

    lax.fori_loop(0, NCH, chunk_body, 0)


def kernel(sequence, token_table):
    tok = sequence.reshape(ROWS)
    table2 = token_table.reshape(1000000 * D // 128, 128)
    out = _embed(tok, table2, jnp.asarray(_PE))
    return out.reshape(BATCH, SEQ, D)
